# Initial kernel scaffold; baseline (speedup 1.0000x reference)
import functools
import jax
import jax.numpy as jnp
from jax import lax
from jax.experimental import pallas as pl
from jax.experimental.pallas import tpu as pltpu

NBLK = 2500


def _mm_kernel(x_ref, w_ref, b_ref, o_ref):
    o_ref[...] = jnp.dot(x_ref[...], w_ref[...],
                         preferred_element_type=jnp.float32) + b_ref[...]


def _node_matmul(x, W, b):
    N, Din = x.shape
    Dout = W.shape[0]
    grid = (N // NBLK,)
    return pl.pallas_call(
        _mm_kernel,
        grid=grid,
        in_specs=[
            pl.BlockSpec((NBLK, Din), lambda i: (i, 0)),
            pl.BlockSpec((Din, Dout), lambda i: (0, 0)),
            pl.BlockSpec((1, Dout), lambda i: (0, 0)),
        ],
        out_specs=pl.BlockSpec((NBLK, Dout), lambda i: (i, 0)),
        out_shape=jax.ShapeDtypeStruct((N, Dout), jnp.float32),
    )(x, W.T, b[None, :])


def _epilogue_kernel(outsum_ref, den_ref, skip_ref, o_ref):
    den = den_ref[...]
    o_ref[...] = jnp.tanh(outsum_ref[...] / (den + 1e-16) + skip_ref[...])


def _layer_epilogue(outsum, den, skip):
    N, D = outsum.shape
    grid = (N // NBLK,)
    return pl.pallas_call(
        _epilogue_kernel,
        grid=grid,
        in_specs=[
            pl.BlockSpec((NBLK, D), lambda i: (i, 0)),
            pl.BlockSpec((NBLK, 1), lambda i: (i, 0)),
            pl.BlockSpec((NBLK, D), lambda i: (i, 0)),
        ],
        out_specs=pl.BlockSpec((NBLK, D), lambda i: (i, 0)),
        out_shape=jax.ShapeDtypeStruct((N, D), jnp.float32),
    )(outsum, den, skip)


def _head_kernel(cat_ref, w1_ref, b1_ref, oh_ref, cnt_ref, sums_ref):
    i = pl.program_id(0)
    z = jnp.dot(cat_ref[...], w1_ref[...],
                preferred_element_type=jnp.float32) + b1_ref[...]
    onehot = oh_ref[...]
    part = jnp.dot(onehot, z, preferred_element_type=jnp.float32)
    cpart = jnp.sum(onehot, axis=1, keepdims=True)

    @pl.when(i == 0)
    def _():
        sums_ref[...] = jnp.zeros_like(sums_ref)
        cnt_ref[...] = jnp.zeros_like(cnt_ref)

    sums_ref[...] += part
    cnt_ref[...] += cpart


def _pool_head(cat, batch, W1, b1, G):
    N = cat.shape[0]
    onehot = (batch[None, :] == jnp.arange(G, dtype=jnp.int32)[:, None])
    onehot = onehot.astype(jnp.float32)
    grid = (N // NBLK,)
    cnt, sums = pl.pallas_call(
        _head_kernel,
        grid=grid,
        in_specs=[
            pl.BlockSpec((NBLK, 128), lambda i: (i, 0)),
            pl.BlockSpec((128, 32), lambda i: (0, 0)),
            pl.BlockSpec((1, 32), lambda i: (0, 0)),
            pl.BlockSpec((G, NBLK), lambda i: (0, i)),
        ],
        out_specs=[
            pl.BlockSpec((G, 1), lambda i: (0, 0)),
            pl.BlockSpec((G, 32), lambda i: (0, 0)),
        ],
        out_shape=[
            jax.ShapeDtypeStruct((G, 1), jnp.float32),
            jax.ShapeDtypeStruct((G, 32), jnp.float32),
        ],
    )(cat, W1.T, b1[None, :], onehot)
    return sums, cnt


def _mlp_kernel(pooled_ref, w2_ref, b2_ref, w3_ref, b3_ref, o_ref):
    z = jnp.dot(pooled_ref[...], w2_ref[...],
                preferred_element_type=jnp.float32) + b2_ref[...]
    z = jnp.maximum(z, 0.0)
    z = jnp.dot(z, w3_ref[...], preferred_element_type=jnp.float32) + b3_ref[...]
    z = z - jnp.max(z, axis=-1, keepdims=True)
    o_ref[...] = z - jnp.log(jnp.sum(jnp.exp(z), axis=-1, keepdims=True))


def _final_mlp(pooled, W2, b2, W3, b3):
    G = pooled.shape[0]
    return pl.pallas_call(
        _mlp_kernel,
        in_specs=[
            pl.BlockSpec((G, 32), lambda: (0, 0)),
            pl.BlockSpec((32, 128), lambda: (0, 0)),
            pl.BlockSpec((1, 128), lambda: (0, 0)),
            pl.BlockSpec((128, 10), lambda: (0, 0)),
            pl.BlockSpec((1, 10), lambda: (0, 0)),
        ],
        out_specs=pl.BlockSpec((G, 10), lambda: (0, 0)),
        out_shape=jax.ShapeDtypeStruct((G, 10), jnp.float32),
    )(pooled, W2.T, b2[None, :], W3.T, b3[None, :])


def _edge_phase(q, k, v, src, dst, n):
    alpha = jnp.sum(q[dst] * k[src], axis=-1) / jnp.sqrt(32.0)
    amax = jax.ops.segment_max(alpha, dst, num_segments=n)
    amax = jnp.where(jnp.isfinite(amax), amax, 0.0)
    ex = jnp.exp(alpha - amax[dst])
    den = jax.ops.segment_sum(ex, dst, num_segments=n)
    outsum = jax.ops.segment_sum(v[src] * ex[:, None], dst, num_segments=n)
    return outsum, den[:, None]


@jax.jit
def kernel(x, edge_index, batch, params):
    n = x.shape[0]
    G = 64
    src, dst = edge_index[0], edge_index[1]
    h_src, h_dst = x[:, :32], x[:, 32:]
    states = []
    for li, p in enumerate(params['convs']):
        if li > 0:
            qkvs = _node_matmul(
                h_src,
                jnp.concatenate([p['Wq'], p['Wk'], p['Wv'], p['Ws']], axis=0),
                jnp.concatenate([p['bq'], p['bk'], p['bv'], p['bs']], axis=0))
            q, kk, v, s = (qkvs[:, :32], qkvs[:, 32:64],
                           qkvs[:, 64:96], qkvs[:, 96:])
        else:
            qs = _node_matmul(h_dst,
                              jnp.concatenate([p['Wq'], p['Ws']], 0),
                              jnp.concatenate([p['bq'], p['bs']], 0))
            kv = _node_matmul(h_src,
                              jnp.concatenate([p['Wk'], p['Wv']], 0),
                              jnp.concatenate([p['bk'], p['bv']], 0))
            q, s = qs[:, :32], qs[:, 32:]
            kk, v = kv[:, :32], kv[:, 32:]
        outsum, den = _edge_phase(q, kk, v, src, dst, n)
        h = _layer_epilogue(outsum, den, s)
        states.append(h)
        h_src = h_dst = h
    cat = jnp.concatenate(states, axis=1)
    sums, cnt = _pool_head(cat, batch, params['W1'], params['b1'], G)
    pooled = sums / jnp.clip(cnt, 1.0)
    return _final_mlp(pooled, params['W2'], params['b2'],
                      params['W3'], params['b3'])


# speedup vs baseline: 1.3559x; 1.3559x over previous
import functools
import jax
import jax.numpy as jnp
from jax import lax
from jax.experimental import pallas as pl
from jax.experimental.pallas import tpu as pltpu

NBLK = 2000


def _mm_kernel(x_ref, w_ref, b_ref, o_ref):
    o_ref[...] = jnp.dot(x_ref[...], w_ref[...],
                         preferred_element_type=jnp.float32) + b_ref[...]


def _node_matmul(x, W, b):
    N, Din = x.shape
    Dout = W.shape[0]
    grid = (N // NBLK,)
    return pl.pallas_call(
        _mm_kernel,
        grid=grid,
        in_specs=[
            pl.BlockSpec((NBLK, Din), lambda i: (i, 0)),
            pl.BlockSpec((Din, Dout), lambda i: (0, 0)),
            pl.BlockSpec((1, Dout), lambda i: (0, 0)),
        ],
        out_specs=pl.BlockSpec((NBLK, Dout), lambda i: (i, 0)),
        out_shape=jax.ShapeDtypeStruct((N, Dout), jnp.float32),
    )(x, W.T, b[None, :])


def _epilogue_kernel(outsum_ref, den_ref, skip_ref, o_ref):
    den = den_ref[...]
    o_ref[...] = jnp.tanh(outsum_ref[...] / (den + 1e-16) + skip_ref[...])


def _layer_epilogue(outsum, den, skip):
    N, D = outsum.shape
    grid = (N // NBLK,)
    return pl.pallas_call(
        _epilogue_kernel,
        grid=grid,
        in_specs=[
            pl.BlockSpec((NBLK, D), lambda i: (i, 0)),
            pl.BlockSpec((NBLK, 1), lambda i: (i, 0)),
            pl.BlockSpec((NBLK, D), lambda i: (i, 0)),
        ],
        out_specs=pl.BlockSpec((NBLK, D), lambda i: (i, 0)),
        out_shape=jax.ShapeDtypeStruct((N, D), jnp.float32),
    )(outsum, den, skip)


def _head_kernel(cat_ref, w1_ref, b1_ref, batch_ref, cnt_ref, sums_ref):
    i = pl.program_id(0)
    z = jnp.dot(cat_ref[...], w1_ref[...],
                preferred_element_type=jnp.float32) + b1_ref[...]
    g_ids = lax.broadcasted_iota(jnp.int32, (batch_ref.shape[0], 64), 1)
    onehot = (batch_ref[...] == g_ids).astype(jnp.float32)
    part = lax.dot_general(onehot, z, (((0,), (0,)), ((), ())),
                           preferred_element_type=jnp.float32)
    cpart = jnp.sum(onehot, axis=0, keepdims=True)

    @pl.when(i == 0)
    def _():
        sums_ref[...] = jnp.zeros_like(sums_ref)
        cnt_ref[...] = jnp.zeros_like(cnt_ref)

    sums_ref[...] += part
    cnt_ref[...] += cpart


def _pool_head(cat, batch, W1, b1, G):
    N = cat.shape[0]
    grid = (N // NBLK,)
    cnt, sums = pl.pallas_call(
        _head_kernel,
        grid=grid,
        in_specs=[
            pl.BlockSpec((NBLK, 128), lambda i: (i, 0)),
            pl.BlockSpec((128, 32), lambda i: (0, 0)),
            pl.BlockSpec((1, 32), lambda i: (0, 0)),
            pl.BlockSpec((NBLK, 1), lambda i: (i, 0)),
        ],
        out_specs=[
            pl.BlockSpec((1, G), lambda i: (0, 0)),
            pl.BlockSpec((G, 32), lambda i: (0, 0)),
        ],
        out_shape=[
            jax.ShapeDtypeStruct((1, G), jnp.float32),
            jax.ShapeDtypeStruct((G, 32), jnp.float32),
        ],
    )(cat, W1.T, b1[None, :], batch[:, None])
    return sums, cnt[0][:, None]


def _mlp_kernel(pooled_ref, w2_ref, b2_ref, w3_ref, b3_ref, o_ref):
    z = jnp.dot(pooled_ref[...], w2_ref[...],
                preferred_element_type=jnp.float32) + b2_ref[...]
    z = jnp.maximum(z, 0.0)
    z = jnp.dot(z, w3_ref[...], preferred_element_type=jnp.float32) + b3_ref[...]
    z = z - jnp.max(z, axis=-1, keepdims=True)
    o_ref[...] = z - jnp.log(jnp.sum(jnp.exp(z), axis=-1, keepdims=True))


def _final_mlp(pooled, W2, b2, W3, b3):
    G = pooled.shape[0]
    return pl.pallas_call(
        _mlp_kernel,
        in_specs=[
            pl.BlockSpec((G, 32), lambda: (0, 0)),
            pl.BlockSpec((32, 128), lambda: (0, 0)),
            pl.BlockSpec((1, 128), lambda: (0, 0)),
            pl.BlockSpec((128, 10), lambda: (0, 0)),
            pl.BlockSpec((1, 10), lambda: (0, 0)),
        ],
        out_specs=pl.BlockSpec((G, 10), lambda: (0, 0)),
        out_shape=jax.ShapeDtypeStruct((G, 10), jnp.float32),
    )(pooled, W2.T, b2[None, :], W3.T, b3[None, :])


def _edge_phase(q, k, v, src, dst, n):
    alpha = jnp.sum(q[dst] * k[src], axis=-1) / jnp.sqrt(32.0)
    amax = jax.ops.segment_max(alpha, dst, num_segments=n)
    amax = jnp.where(jnp.isfinite(amax), amax, 0.0)
    ex = jnp.exp(alpha - amax[dst])
    den = jax.ops.segment_sum(ex, dst, num_segments=n)
    outsum = jax.ops.segment_sum(v[src] * ex[:, None], dst, num_segments=n)
    return outsum, den[:, None]


@jax.jit
def kernel(x, edge_index, batch, params):
    n = x.shape[0]
    G = 64
    src, dst = edge_index[0], edge_index[1]
    h_src, h_dst = x[:, :32], x[:, 32:]
    states = []
    for li, p in enumerate(params['convs']):
        if li > 0:
            qkvs = _node_matmul(
                h_src,
                jnp.concatenate([p['Wq'], p['Wk'], p['Wv'], p['Ws']], axis=0),
                jnp.concatenate([p['bq'], p['bk'], p['bv'], p['bs']], axis=0))
            q, kk, v, s = (qkvs[:, :32], qkvs[:, 32:64],
                           qkvs[:, 64:96], qkvs[:, 96:])
        else:
            qs = _node_matmul(h_dst,
                              jnp.concatenate([p['Wq'], p['Ws']], 0),
                              jnp.concatenate([p['bq'], p['bs']], 0))
            kv = _node_matmul(h_src,
                              jnp.concatenate([p['Wk'], p['Wv']], 0),
                              jnp.concatenate([p['bk'], p['bv']], 0))
            q, s = qs[:, :32], qs[:, 32:]
            kk, v = kv[:, :32], kv[:, 32:]
        outsum, den = _edge_phase(q, kk, v, src, dst, n)
        h = _layer_epilogue(outsum, den, s)
        states.append(h)
        h_src = h_dst = h
    cat = jnp.concatenate(states, axis=1)
    sums, cnt = _pool_head(cat, batch, params['W1'], params['b1'], G)
    pooled = sums / jnp.clip(cnt, 1.0)
    return _final_mlp(pooled, params['W2'], params['b2'],
                      params['W3'], params['b3'])
